# Initial kernel scaffold; baseline (speedup 1.0000x reference)
import jax
import jax.numpy as jnp
from jax.experimental import pallas as pl


def kernel(pos_input, pos_output, neg_v, input_weight, output_weight):
    raise NotImplementedError("write your pallas kernel here")


# speedup vs baseline: 3.4620x; 3.4620x over previous
import functools

import jax
import jax.numpy as jnp
from jax import lax
from jax.experimental import pallas as pl
from jax.experimental.pallas import tpu as pltpu
from jax.experimental.pallas import tpu_sc as plsc

_D = 128
_B = 128
_K = 64
_NC = 2
_NS = 16
_NW = _NC * _NS
_BPW = _B // _NW
_NEG_PW = _B * _K // _NW


def _sc_gather(input_weight, output_weight, idx_in, idx_out, idx_neg):
  mesh = plsc.VectorSubcoreMesh(core_axis_name="c", subcore_axis_name="s")

  @functools.partial(
      pl.kernel,
      mesh=mesh,
      out_type=[
          jax.ShapeDtypeStruct((_B, _D), jnp.float32),
          jax.ShapeDtypeStruct((_B, _D), jnp.float32),
          jax.ShapeDtypeStruct((_B * _K, _D), jnp.float32),
      ],
      scratch_types=[
          pltpu.VMEM((16,), jnp.int32),
          pltpu.VMEM((16,), jnp.int32),
          pltpu.VMEM((2, 128), jnp.int32),
          pltpu.VMEM((16, _D), jnp.float32),
          pltpu.VMEM((16, _D), jnp.float32),
          pltpu.VMEM((_NEG_PW, _D), jnp.float32),
          pltpu.SemaphoreType.DMA,
          pltpu.SemaphoreType.DMA,
          pltpu.SemaphoreType.DMA,
          pltpu.SemaphoreType.DMA,
      ],
  )
  def gather_kernel(iw_hbm, ow_hbm, idx_in_hbm, idx_out_hbm, idx_neg_hbm,
                    emb_in_hbm, emb_out_hbm, emb_neg_hbm,
                    iin_v, iout_v, ineg_v, rin_v, rout_v, rneg_v,
                    s0, s1, s2, s3):
    w = lax.axis_index("s") * _NC + lax.axis_index("c")
    pltpu.sync_copy(idx_in_hbm.at[w], iin_v)
    pltpu.sync_copy(idx_out_hbm.at[w], iout_v)
    pltpu.sync_copy(idx_neg_hbm.at[w], ineg_v)
    c0 = pltpu.async_copy(iw_hbm.at[iin_v], rin_v, s0)
    c1 = pltpu.async_copy(ow_hbm.at[iout_v], rout_v, s1)
    c2 = pltpu.async_copy(ow_hbm.at[ineg_v.at[0]], rneg_v.at[pl.ds(0, 128)], s2)
    c3 = pltpu.async_copy(ow_hbm.at[ineg_v.at[1]], rneg_v.at[pl.ds(128, 128)], s3)
    c0.wait()
    c1.wait()
    c2.wait()
    c3.wait()
    pltpu.sync_copy(rin_v.at[pl.ds(0, _BPW)],
                    emb_in_hbm.at[pl.ds(w * _BPW, _BPW)])
    pltpu.sync_copy(rout_v.at[pl.ds(0, _BPW)],
                    emb_out_hbm.at[pl.ds(w * _BPW, _BPW)])
    pltpu.sync_copy(rneg_v, emb_neg_hbm.at[pl.ds(w * _NEG_PW, _NEG_PW)])

  return gather_kernel(input_weight, output_weight, idx_in, idx_out, idx_neg)


def _logsig(x):
  return jnp.minimum(x, 0.0) - jnp.log1p(jnp.exp(-jnp.abs(x)))


def _tc_body(ei_ref, eo_ref, en_ref, out_ref):
  ei = ei_ref[...]
  eo = eo_ref[...]
  r = jnp.sum(eo, axis=1)
  score = jnp.sum(ei * r[None, :], axis=1)
  pos = -_logsig(score)
  en = en_ref[...].reshape(_B, _K, _D)
  ns = jnp.sum(en * ei[:, None, :], axis=2)
  ns = jnp.clip(ns, -10.0, 10.0)
  t = -jnp.sum(_logsig(-ns), axis=1)
  neg_loss = -jnp.sum(_logsig(-t))
  out_ref[0, 0] = jnp.mean(pos) + neg_loss


def kernel(pos_input, pos_output, neg_v, input_weight, output_weight):
  pi = pos_input.astype(jnp.int32).reshape(_NW, _BPW)
  pi = jnp.concatenate(
      [pi, jnp.broadcast_to(pi[:, :1], (_NW, 16 - _BPW))], axis=1)
  po = pos_output.astype(jnp.int32).reshape(_NW, _BPW)
  po = jnp.concatenate(
      [po, jnp.broadcast_to(po[:, :1], (_NW, 16 - _BPW))], axis=1)
  nv = neg_v.astype(jnp.int32).reshape(_NW, 2, 128)

  emb_in, emb_out, emb_neg = _sc_gather(input_weight, output_weight,
                                        pi, po, nv)

  out = pl.pallas_call(
      _tc_body,
      out_shape=jax.ShapeDtypeStruct((1, 1), jnp.float32),
      out_specs=pl.BlockSpec(memory_space=pltpu.SMEM),
  )(emb_in, emb_out, emb_neg)
  return out[0, 0]
